# baseline (speedup 1.0000x reference)
import jax
import jax.numpy as jnp
from jax import lax
from jax.experimental import pallas as pl
from jax.experimental.pallas import tpu as pltpu

_VOCAB = 1000000
_EMBED = 64
_BLK = 25600
_NBLK = 40
_PAD = _BLK * _NBLK


def _main_body(idx_ref, et_ref, wt_ref, b_ref, logits_ref, logz_ref,
               m_ref, s_ref):
    i = pl.program_id(0)

    @pl.when(i == 0)
    def _init():
        m_ref[...] = jnp.full((1, 1), -jnp.inf, jnp.float32)
        s_ref[...] = jnp.zeros((1, 1), jnp.float32)

    lane = idx_ref[0] % 128
    lanemask = lax.broadcasted_iota(jnp.int32, (_EMBED, 128), 1) == lane
    esel = jnp.where(lanemask, et_ref[...], jnp.zeros_like(et_ref[...]))
    e_col = jnp.sum(esel, axis=1, keepdims=True)

    wt = wt_ref[...]
    logits = jnp.sum(wt * e_col, axis=0, keepdims=True)
    logits = logits + b_ref[...].reshape(1, _BLK)

    col = lax.broadcasted_iota(jnp.int32, (1, _BLK), 1) + i * _BLK
    logits = jnp.where(col < _VOCAB, logits, -jnp.inf)
    logits_ref[...] = logits

    bm = jnp.max(logits, axis=(0, 1), keepdims=True)
    m_old = m_ref[...]
    m_new = jnp.maximum(m_old, bm)
    se = jnp.sum(jnp.exp(logits - m_new), axis=(0, 1), keepdims=True)
    s_ref[...] = s_ref[...] * jnp.exp(m_old - m_new) + se
    m_ref[...] = m_new

    @pl.when(i == _NBLK - 1)
    def _fin():
        logz_ref[...] = m_ref[...] + jnp.log(s_ref[...])


def _final_body(logits_ref, logz_ref, out_ref):
    out_ref[...] = logits_ref[...] - logz_ref[...]


def kernel(indices, emb_table, W, b):
    idx = indices.astype(jnp.int32)
    et = emb_table.T
    wt = W.T

    grid_spec = pltpu.PrefetchScalarGridSpec(
        num_scalar_prefetch=1,
        grid=(_NBLK,),
        in_specs=[
            pl.BlockSpec((_EMBED, 128), lambda i, idx_ref: (0, idx_ref[0] // 128)),
            pl.BlockSpec((_EMBED, _BLK), lambda i, idx_ref: (0, i)),
            pl.BlockSpec((_BLK,), lambda i, idx_ref: (i,)),
        ],
        out_specs=[
            pl.BlockSpec((1, _BLK), lambda i, idx_ref: (0, i)),
            pl.BlockSpec((1, 1), lambda i, idx_ref: (0, 0)),
        ],
        scratch_shapes=[
            pltpu.VMEM((1, 1), jnp.float32),
            pltpu.VMEM((1, 1), jnp.float32),
        ],
    )
    logits, logz = pl.pallas_call(
        _main_body,
        grid_spec=grid_spec,
        out_shape=[
            jax.ShapeDtypeStruct((1, _PAD), jnp.float32),
            jax.ShapeDtypeStruct((1, 1), jnp.float32),
        ],
    )(idx, et, wt, b)

    out = pl.pallas_call(
        _final_body,
        grid=(_NBLK,),
        in_specs=[
            pl.BlockSpec((1, _BLK), lambda i: (0, i)),
            pl.BlockSpec((1, 1), lambda i: (0, 0)),
        ],
        out_specs=pl.BlockSpec((1, _BLK), lambda i: (0, i)),
        out_shape=jax.ShapeDtypeStruct((1, _VOCAB), jnp.float32),
    )(logits, logz)
    return out


# speedup vs baseline: 1.1937x; 1.1937x over previous
import jax
import jax.numpy as jnp
from jax import lax
from jax.experimental import pallas as pl
from jax.experimental.pallas import tpu as pltpu

_VOCAB = 1000000
_EMBED = 64
_BLK = 40960
_NBLK = 25
_PAD = _BLK * _NBLK


def _main_body(idx_ref, et_ref, wt_ref, b_ref, logits_ref, logz_ref,
               m_ref, s_ref):
    i = pl.program_id(0)

    @pl.when(i == 0)
    def _init():
        m_ref[...] = jnp.full((1, 1), -jnp.inf, jnp.float32)
        s_ref[...] = jnp.zeros((1, 1), jnp.float32)

    lane = idx_ref[0] % 128
    lanemask = lax.broadcasted_iota(jnp.int32, (_EMBED, 128), 1) == lane
    esel = jnp.where(lanemask, et_ref[...], jnp.zeros_like(et_ref[...]))
    e_col = jnp.sum(esel, axis=1, keepdims=True)

    wt = wt_ref[...]
    logits = jnp.sum(wt * e_col, axis=0, keepdims=True)
    logits = logits + b_ref[...].reshape(1, _BLK)

    col = lax.broadcasted_iota(jnp.int32, (1, _BLK), 1) + i * _BLK
    logits = jnp.where(col < _VOCAB, logits, -jnp.inf)
    logits_ref[...] = logits

    bm = jnp.max(logits, axis=(0, 1), keepdims=True)
    m_old = m_ref[...]
    m_new = jnp.maximum(m_old, bm)
    se = jnp.sum(jnp.exp(logits - m_new), axis=(0, 1), keepdims=True)
    s_ref[...] = s_ref[...] * jnp.exp(m_old - m_new) + se
    m_ref[...] = m_new

    @pl.when(i == _NBLK - 1)
    def _fin():
        logz_ref[...] = m_ref[...] + jnp.log(s_ref[...])


def _final_body(logits_ref, logz_ref, out_ref):
    out_ref[...] = logits_ref[...] - logz_ref[...]


def kernel(indices, emb_table, W, b):
    idx = indices.astype(jnp.int32)
    et = emb_table.T
    wt = W.T

    grid_spec = pltpu.PrefetchScalarGridSpec(
        num_scalar_prefetch=1,
        grid=(_NBLK,),
        in_specs=[
            pl.BlockSpec((_EMBED, 128), lambda i, idx_ref: (0, idx_ref[0] // 128)),
            pl.BlockSpec((_EMBED, _BLK), lambda i, idx_ref: (0, i)),
            pl.BlockSpec((_BLK,), lambda i, idx_ref: (i,)),
        ],
        out_specs=[
            pl.BlockSpec((1, _BLK), lambda i, idx_ref: (0, i)),
            pl.BlockSpec((1, 1), lambda i, idx_ref: (0, 0)),
        ],
        scratch_shapes=[
            pltpu.VMEM((1, 1), jnp.float32),
            pltpu.VMEM((1, 1), jnp.float32),
        ],
    )
    logits, logz = pl.pallas_call(
        _main_body,
        grid_spec=grid_spec,
        out_shape=[
            jax.ShapeDtypeStruct((1, _PAD), jnp.float32),
            jax.ShapeDtypeStruct((1, 1), jnp.float32),
        ],
    )(idx, et, wt, b)

    out = pl.pallas_call(
        _final_body,
        grid=(_NBLK,),
        in_specs=[
            pl.BlockSpec((1, _BLK), lambda i: (0, i)),
            pl.BlockSpec((1, 1), lambda i: (0, 0)),
        ],
        out_specs=pl.BlockSpec((1, _BLK), lambda i: (0, i)),
        out_shape=jax.ShapeDtypeStruct((1, _VOCAB), jnp.float32),
    )(logits, logz)
    return out


# speedup vs baseline: 1.3312x; 1.1152x over previous
import jax
import jax.numpy as jnp
from jax import lax
from jax.experimental import pallas as pl
from jax.experimental.pallas import tpu as pltpu

_VOCAB = 1000000
_EMBED = 64
_BLK = 40960
_NBLK = 25
_PAD = _BLK * _NBLK


def _main_body(idx_ref, et_ref, wt_ref, b_ref, logits_ref, logz_ref,
               m_ref, s_ref):
    i = pl.program_id(0)

    @pl.when(i == 0)
    def _init():
        m_ref[...] = jnp.full((1, 1), -jnp.inf, jnp.float32)
        s_ref[...] = jnp.zeros((1, 1), jnp.float32)

    lane = idx_ref[0] % 128
    lanemask = lax.broadcasted_iota(jnp.int32, (_EMBED, 128), 1) == lane
    esel = jnp.where(lanemask, et_ref[...], jnp.zeros_like(et_ref[...]))
    e_col = jnp.sum(esel, axis=1, keepdims=True)

    wt = wt_ref[...]
    logits = jnp.sum(wt * e_col, axis=0, keepdims=True)
    logits = logits + b_ref[...].reshape(1, _BLK)

    col = lax.broadcasted_iota(jnp.int32, (1, _BLK), 1) + i * _BLK
    logits = jnp.where(col < _VOCAB, logits, -jnp.inf)
    logits_ref[...] = logits

    bm = jnp.max(logits, axis=(0, 1), keepdims=True)
    m_old = m_ref[...]
    m_new = jnp.maximum(m_old, bm)
    se = jnp.sum(jnp.exp(logits - m_new), axis=(0, 1), keepdims=True)
    s_ref[...] = s_ref[...] * jnp.exp(m_old - m_new) + se
    m_ref[...] = m_new

    @pl.when(i == _NBLK - 1)
    def _fin():
        logz_ref[...] = m_ref[...] + jnp.log(s_ref[...])


def _final_body(logits_ref, logz_ref, out_ref):
    out_ref[...] = logits_ref[:, :_VOCAB] - logz_ref[...]


def kernel(indices, emb_table, W, b):
    idx = indices.astype(jnp.int32)
    et = emb_table.T
    wt = W.T

    grid_spec = pltpu.PrefetchScalarGridSpec(
        num_scalar_prefetch=1,
        grid=(_NBLK,),
        in_specs=[
            pl.BlockSpec((_EMBED, 128), lambda i, idx_ref: (0, idx_ref[0] // 128)),
            pl.BlockSpec((_EMBED, _BLK), lambda i, idx_ref: (0, i)),
            pl.BlockSpec((_BLK,), lambda i, idx_ref: (i,)),
        ],
        out_specs=[
            pl.BlockSpec((1, _BLK), lambda i, idx_ref: (0, i)),
            pl.BlockSpec((1, 1), lambda i, idx_ref: (0, 0)),
        ],
        scratch_shapes=[
            pltpu.VMEM((1, 1), jnp.float32),
            pltpu.VMEM((1, 1), jnp.float32),
        ],
    )
    logits, logz = pl.pallas_call(
        _main_body,
        grid_spec=grid_spec,
        out_shape=[
            jax.ShapeDtypeStruct((1, _PAD), jnp.float32),
            jax.ShapeDtypeStruct((1, 1), jnp.float32),
        ],
    )(idx, et, wt, b)

    out = pl.pallas_call(
        _final_body,
        in_specs=[
            pl.BlockSpec((1, _PAD), lambda: (0, 0)),
            pl.BlockSpec((1, 1), lambda: (0, 0)),
        ],
        out_specs=pl.BlockSpec((1, _VOCAB), lambda: (0, 0)),
        out_shape=jax.ShapeDtypeStruct((1, _VOCAB), jnp.float32),
    )(logits, logz)
    return out


# speedup vs baseline: 1.4375x; 1.0799x over previous
import jax
import jax.numpy as jnp
from jax import lax
from jax.experimental import pallas as pl
from jax.experimental.pallas import tpu as pltpu

_VOCAB = 1000000
_EMBED = 64
_BLK = 40960
_NBLK = 25
_PAD = _BLK * _NBLK


def _main_body(idx_ref, et_ref, wt_ref, b_ref, logits_ref, logz_ref,
               m_ref, s_ref):
    i = pl.program_id(0)

    @pl.when(i == 0)
    def _init():
        m_ref[...] = jnp.full((1, 1), -jnp.inf, jnp.float32)
        s_ref[...] = jnp.zeros((1, 1), jnp.float32)

    lane = idx_ref[0] % 128
    lanemask = lax.broadcasted_iota(jnp.int32, (_EMBED, 128), 1) == lane
    esel = jnp.where(lanemask, et_ref[...], jnp.zeros_like(et_ref[...]))
    e_col = jnp.sum(esel, axis=1, keepdims=True)

    e_row = e_col.T.astype(jnp.bfloat16)
    wt = wt_ref[...].astype(jnp.bfloat16)
    logits = lax.dot_general(e_row, wt, (((1,), (0,)), ((), ())),
                             preferred_element_type=jnp.float32)
    logits = logits + b_ref[...].reshape(1, _BLK)

    col = lax.broadcasted_iota(jnp.int32, (1, _BLK), 1) + i * _BLK
    logits = jnp.where(col < _VOCAB, logits, -jnp.inf)
    logits_ref[...] = logits

    bm = jnp.max(logits, axis=(0, 1), keepdims=True)
    m_old = m_ref[...]
    m_new = jnp.maximum(m_old, bm)
    se = jnp.sum(jnp.exp(logits - m_new), axis=(0, 1), keepdims=True)
    s_ref[...] = s_ref[...] * jnp.exp(m_old - m_new) + se
    m_ref[...] = m_new

    @pl.when(i == _NBLK - 1)
    def _fin():
        logz_ref[...] = m_ref[...] + jnp.log(s_ref[...])


def _final_body(logits_ref, logz_ref, out_ref):
    out_ref[...] = logits_ref[:, :_VOCAB] - logz_ref[...]


def kernel(indices, emb_table, W, b):
    idx = indices.astype(jnp.int32)
    et = emb_table.T
    wt = W.T

    grid_spec = pltpu.PrefetchScalarGridSpec(
        num_scalar_prefetch=1,
        grid=(_NBLK,),
        in_specs=[
            pl.BlockSpec((_EMBED, 128), lambda i, idx_ref: (0, idx_ref[0] // 128)),
            pl.BlockSpec((_EMBED, _BLK), lambda i, idx_ref: (0, i)),
            pl.BlockSpec((_BLK,), lambda i, idx_ref: (i,)),
        ],
        out_specs=[
            pl.BlockSpec((1, _BLK), lambda i, idx_ref: (0, i)),
            pl.BlockSpec((1, 1), lambda i, idx_ref: (0, 0)),
        ],
        scratch_shapes=[
            pltpu.VMEM((1, 1), jnp.float32),
            pltpu.VMEM((1, 1), jnp.float32),
        ],
    )
    logits, logz = pl.pallas_call(
        _main_body,
        grid_spec=grid_spec,
        out_shape=[
            jax.ShapeDtypeStruct((1, _PAD), jnp.float32),
            jax.ShapeDtypeStruct((1, 1), jnp.float32),
        ],
    )(idx, et, wt, b)

    out = pl.pallas_call(
        _final_body,
        in_specs=[
            pl.BlockSpec((1, _PAD), lambda: (0, 0)),
            pl.BlockSpec((1, 1), lambda: (0, 0)),
        ],
        out_specs=pl.BlockSpec((1, _VOCAB), lambda: (0, 0)),
        out_shape=jax.ShapeDtypeStruct((1, _VOCAB), jnp.float32),
    )(logits, logz)
    return out


# speedup vs baseline: 1.4464x; 1.0062x over previous
import jax
import jax.numpy as jnp
from jax import lax
from jax.experimental import pallas as pl
from jax.experimental.pallas import tpu as pltpu

_VOCAB = 1000000
_EMBED = 64
_BLK = 51200
_NBLK = 20
_PAD = _BLK * _NBLK


def _main_body(idx_ref, et_ref, wt_ref, b_ref, logits_ref, logz_ref,
               m_ref, s_ref):
    i = pl.program_id(0)

    @pl.when(i == 0)
    def _init():
        m_ref[...] = jnp.full((1, 1), -jnp.inf, jnp.float32)
        s_ref[...] = jnp.zeros((1, 1), jnp.float32)

    lane = idx_ref[0] % 128
    lanemask = lax.broadcasted_iota(jnp.int32, (_EMBED, 128), 1) == lane
    esel = jnp.where(lanemask, et_ref[...], jnp.zeros_like(et_ref[...]))
    e_col = jnp.sum(esel, axis=1, keepdims=True)

    e_row = e_col.T.astype(jnp.bfloat16)
    wt = wt_ref[...].astype(jnp.bfloat16)
    logits = lax.dot_general(e_row, wt, (((1,), (0,)), ((), ())),
                             preferred_element_type=jnp.float32)
    logits = logits + b_ref[...].reshape(1, _BLK)

    col = lax.broadcasted_iota(jnp.int32, (1, _BLK), 1) + i * _BLK
    logits = jnp.where(col < _VOCAB, logits, -jnp.inf)
    logits_ref[...] = logits

    bm = jnp.max(logits, axis=(0, 1), keepdims=True)
    m_old = m_ref[...]
    m_new = jnp.maximum(m_old, bm)
    se = jnp.sum(jnp.exp(logits - m_new), axis=(0, 1), keepdims=True)
    s_ref[...] = s_ref[...] * jnp.exp(m_old - m_new) + se
    m_ref[...] = m_new

    @pl.when(i == _NBLK - 1)
    def _fin():
        logz_ref[...] = m_ref[...] + jnp.log(s_ref[...])


def _final_body(logits_ref, logz_ref, out_ref):
    out_ref[...] = logits_ref[:, :_VOCAB] - logz_ref[...]


def kernel(indices, emb_table, W, b):
    idx = indices.astype(jnp.int32)
    et = emb_table.T
    wt = W.T

    grid_spec = pltpu.PrefetchScalarGridSpec(
        num_scalar_prefetch=1,
        grid=(_NBLK,),
        in_specs=[
            pl.BlockSpec((_EMBED, 128), lambda i, idx_ref: (0, idx_ref[0] // 128)),
            pl.BlockSpec((_EMBED, _BLK), lambda i, idx_ref: (0, i)),
            pl.BlockSpec((_BLK,), lambda i, idx_ref: (i,)),
        ],
        out_specs=[
            pl.BlockSpec((1, _BLK), lambda i, idx_ref: (0, i)),
            pl.BlockSpec((1, 1), lambda i, idx_ref: (0, 0)),
        ],
        scratch_shapes=[
            pltpu.VMEM((1, 1), jnp.float32),
            pltpu.VMEM((1, 1), jnp.float32),
        ],
    )
    logits, logz = pl.pallas_call(
        _main_body,
        grid_spec=grid_spec,
        out_shape=[
            jax.ShapeDtypeStruct((1, _PAD), jnp.float32),
            jax.ShapeDtypeStruct((1, 1), jnp.float32),
        ],
    )(idx, et, wt, b)

    out = pl.pallas_call(
        _final_body,
        in_specs=[
            pl.BlockSpec((1, _PAD), lambda: (0, 0)),
            pl.BlockSpec((1, 1), lambda: (0, 0)),
        ],
        out_specs=pl.BlockSpec((1, _VOCAB), lambda: (0, 0)),
        out_shape=jax.ShapeDtypeStruct((1, _VOCAB), jnp.float32),
    )(logits, logz)
    return out
